# baseline (speedup 1.0000x reference)
import functools

import jax
import jax.numpy as jnp
from jax import lax
from jax.experimental import pallas as pl
from jax.experimental.pallas import tpu as pltpu
from jax.experimental.pallas import tpu_sc as plsc

_VOCAB = 1000
_NTOK = 1024 * 50


def _table_body(embed_ref, w_ref, b_ref, t_ref):
    t_ref[...] = (
        lax.dot_general(
            embed_ref[...],
            w_ref[...],
            dimension_numbers=(((1,), (1,)), ((), ())),
            preferred_element_type=jnp.float32,
        )
        + b_ref[...]
    )


def _make_table():
    return pl.pallas_call(
        _table_body,
        out_shape=jax.ShapeDtypeStruct((_VOCAB, _VOCAB), jnp.float32),
    )


def _make_gather():
    info = plsc.get_sparse_core_info()
    nc, ns = info.num_cores, info.num_subcores
    nw = nc * ns
    b_per_w = _NTOK // nw
    g = 80
    n_chunks = b_per_w // g
    mesh = plsc.VectorSubcoreMesh(core_axis_name="c", subcore_axis_name="s")

    @functools.partial(
        pl.kernel,
        mesh=mesh,
        compiler_params=pltpu.CompilerParams(use_tc_tiling_on_sc=False),
        out_type=jax.ShapeDtypeStruct((_NTOK, _VOCAB), jnp.float32),
        scratch_types=[
            pltpu.VMEM((b_per_w,), jnp.int32),
            pltpu.VMEM((g, _VOCAB), jnp.float32),
            pltpu.SemaphoreType.DMA,
        ],
    )
    def gather(table_hbm, idx_hbm, out_hbm, idx_v, rows_v, sem):
        wid = lax.axis_index("s") * nc + lax.axis_index("c")
        base = wid * b_per_w
        pltpu.sync_copy(idx_hbm.at[pl.ds(base, b_per_w)], idx_v)

        def body(c, _):
            row0 = c * g
            pltpu.async_copy(
                table_hbm.at[idx_v.at[pl.ds(row0, g)]], rows_v, sem
            ).wait()
            pltpu.sync_copy(rows_v, out_hbm.at[pl.ds(base + row0, g)])
            return _

        lax.fori_loop(0, n_chunks, body, None)

    return gather


_table = _make_table()
_gather = _make_gather()


def kernel(x, embed, W, b):
    t = _table(embed, W, b.reshape(1, _VOCAB))
    idx = x.reshape(-1).astype(jnp.int32)
    out = _gather(t, idx)
    return out.reshape(x.shape[0], x.shape[1], _VOCAB)


# speedup vs baseline: 1.6251x; 1.6251x over previous
import functools

import jax
import jax.numpy as jnp
from jax import lax
from jax.experimental import pallas as pl
from jax.experimental.pallas import tpu as pltpu
from jax.experimental.pallas import tpu_sc as plsc

_VOCAB = 1000
_NPANEL = 8
_TAIL = _VOCAB - 7 * 128
_B = 1024
_L = 50


def _table_body(embed_ref, w_ref, b_ref, *t_refs):
    t = (
        lax.dot_general(
            embed_ref[...],
            w_ref[...],
            dimension_numbers=(((1,), (1,)), ((), ())),
            preferred_element_type=jnp.float32,
        )
        + b_ref[...]
    )
    for k, t_ref in enumerate(t_refs):
        t_ref[...] = t[:, 128 * k : 128 * (k + 1)]


def _make_table():
    return pl.pallas_call(
        _table_body,
        out_shape=tuple(
            jax.ShapeDtypeStruct((_VOCAB, 128), jnp.float32)
            for _ in range(_NPANEL)
        ),
    )


def _make_gather():
    info = plsc.get_sparse_core_info()
    nc, ns = info.num_cores, info.num_subcores
    nw = nc * ns
    b_per_w = _B // nw
    mesh = plsc.VectorSubcoreMesh(core_axis_name="c", subcore_axis_name="s")

    col_offs = [0, 16, 32, 48, 64, 80, 88]

    @functools.partial(
        pl.kernel,
        mesh=mesh,
        out_type=jax.ShapeDtypeStruct((_B, _L, _VOCAB), jnp.float32),
        scratch_types=[
            pltpu.VMEM((b_per_w, _L), jnp.int32),
            [pltpu.VMEM((_L, 128), jnp.float32) for _ in range(_NPANEL)],
            pltpu.VMEM((_L, _TAIL), jnp.float32),
            pltpu.SemaphoreType.DMA,
        ],
    )
    def gather(*refs):
        t_hbm = refs[:_NPANEL]
        idx_hbm, out_hbm, idx_v, panels, tail_v, sem = refs[_NPANEL:]
        wid = lax.axis_index("s") * nc + lax.axis_index("c")
        base = wid * b_per_w
        pltpu.sync_copy(idx_hbm.at[pl.ds(base, b_per_w)], idx_v)

        def body(i, _):
            copies = [
                pltpu.async_copy(t_hbm[k].at[idx_v.at[i]], panels[k], sem)
                for k in range(_NPANEL)
            ]
            for c in copies:
                c.wait()

            def repack(l, __):
                for c in col_offs:
                    tail_v[l, pl.ds(c, 16)] = panels[_NPANEL - 1][l, pl.ds(c, 16)]
                return __

            lax.fori_loop(0, _L, repack, None)
            for k in range(_NPANEL - 1):
                pltpu.sync_copy(
                    panels[k], out_hbm.at[base + i, :, pl.ds(128 * k, 128)]
                )
            pltpu.sync_copy(
                tail_v, out_hbm.at[base + i, :, pl.ds(128 * (_NPANEL - 1), _TAIL)]
            )
            return _

        lax.fori_loop(0, b_per_w, body, None)

    return gather


_table = _make_table()
_gather = _make_gather()


def kernel(x, embed, W, b):
    w_pad = jnp.pad(W, ((0, 24), (0, 0)))
    b_pad = jnp.pad(b, (0, 24)).reshape(1, 1024)
    t_parts = _table(embed, w_pad, b_pad)
    out = _gather(*t_parts, x.astype(jnp.int32))
    return out


# speedup vs baseline: 1.7607x; 1.0834x over previous
import functools

import jax
import jax.numpy as jnp
from jax import lax
from jax.experimental import pallas as pl
from jax.experimental.pallas import tpu as pltpu
from jax.experimental.pallas import tpu_sc as plsc

_VOCAB = 1000
_NPANEL = 8
_TAIL = _VOCAB - 7 * 128
_B = 1024
_L = 50


def _table_body(embed_ref, w_ref, b_ref, *t_refs):
    t = (
        lax.dot_general(
            embed_ref[...],
            w_ref[...],
            dimension_numbers=(((1,), (1,)), ((), ())),
            preferred_element_type=jnp.float32,
        )
        + b_ref[...]
    )
    for k, t_ref in enumerate(t_refs):
        t_ref[...] = t[:, 128 * k : 128 * (k + 1)]


def _make_table():
    return pl.pallas_call(
        _table_body,
        out_shape=tuple(
            jax.ShapeDtypeStruct((_VOCAB, 128), jnp.float32)
            for _ in range(_NPANEL)
        ),
    )


def _make_gather():
    info = plsc.get_sparse_core_info()
    nc, ns = info.num_cores, info.num_subcores
    nw = nc * ns
    b_per_w = _B // nw
    mesh = plsc.VectorSubcoreMesh(core_axis_name="c", subcore_axis_name="s")

    col_offs = [0, 16, 32, 48, 64, 80, 88]

    @functools.partial(
        pl.kernel,
        mesh=mesh,
        out_type=jax.ShapeDtypeStruct((_B, _L, _VOCAB), jnp.float32),
        scratch_types=[
            pltpu.VMEM((b_per_w, _L), jnp.int32),
            [pltpu.VMEM_SHARED((_VOCAB, 128), jnp.float32) for _ in range(_NPANEL - 1)],
            [pltpu.VMEM((_L, 128), jnp.float32) for _ in range(_NPANEL)],
            pltpu.VMEM((_L, _TAIL), jnp.float32),
            pltpu.SemaphoreType.DMA,
            pltpu.SemaphoreType.DMA,
        ],
    )
    def gather(*refs):
        t_hbm = refs[:_NPANEL]
        idx_hbm, out_hbm, idx_v, t_sp, panels, tail_v, sem, sem_hbm = refs[_NPANEL:]
        sid = lax.axis_index("s")
        wid = sid * nc + lax.axis_index("c")
        base = wid * b_per_w

        n_chunk = (_VOCAB + 47) // 48
        jobs = [(k, c) for k in range(_NPANEL - 1) for c in range(n_chunk)]
        for j, (k, c) in enumerate(jobs):
            r0 = 48 * c
            rows = 48 if c < n_chunk - 1 else _VOCAB - 48 * (n_chunk - 1)

            @pl.when(sid == (j % ns))
            def _(k=k, r0=r0, rows=rows, buf=panels[j % _NPANEL]):
                pltpu.sync_copy(t_hbm[k].at[pl.ds(r0, rows)], buf.at[pl.ds(0, rows)])
                pltpu.sync_copy(buf.at[pl.ds(0, rows)], t_sp[k].at[pl.ds(r0, rows)])

        pltpu.sync_copy(idx_hbm.at[pl.ds(base, b_per_w)], idx_v)
        plsc.subcore_barrier()

        def body(i, _):
            copies = [
                pltpu.async_copy(t_sp[k].at[idx_v.at[i]], panels[k], sem)
                for k in range(_NPANEL - 1)
            ]
            copies.append(
                pltpu.async_copy(
                    t_hbm[_NPANEL - 1].at[idx_v.at[i]], panels[_NPANEL - 1], sem_hbm
                )
            )
            for c in copies:
                c.wait()

            def repack(l, __):
                for c in col_offs:
                    tail_v[l, pl.ds(c, 16)] = panels[_NPANEL - 1][l, pl.ds(c, 16)]
                return __

            lax.fori_loop(0, _L, repack, None)
            for k in range(_NPANEL - 1):
                pltpu.sync_copy(
                    panels[k], out_hbm.at[base + i, :, pl.ds(128 * k, 128)]
                )
            pltpu.sync_copy(
                tail_v, out_hbm.at[base + i, :, pl.ds(128 * (_NPANEL - 1), _TAIL)]
            )
            return _

        lax.fori_loop(0, b_per_w, body, None)

    return gather


_table = _make_table()
_gather = _make_gather()


def kernel(x, embed, W, b):
    w_pad = jnp.pad(W, ((0, 24), (0, 0)))
    b_pad = jnp.pad(b, (0, 24)).reshape(1, 1024)
    t_parts = _table(embed, w_pad, b_pad)
    out = _gather(*t_parts, x.astype(jnp.int32))
    return out
